# baseline (speedup 1.0000x reference)
import functools

import jax
import jax.numpy as jnp
from jax import lax
from jax.experimental import pallas as pl
from jax.experimental.pallas import tpu as pltpu
from jax.experimental.pallas import tpu_sc as plsc

N = 10000
E = 320000
G = 128
Q = 80
NS = 16
CAU_GAMMA = 0.4

_SC_PARAMS = pltpu.CompilerParams(use_tc_tiling_on_sc=False,
                                  needs_layout_passes=False)


def _segsum_sc(ts, src, dst, w=None, invert=False, Dq=Q, C=400):
    e_per = E // NS
    nchunk = e_per // C
    assert nchunk % 2 == 0
    RPS = 624
    TAIL = N - RPS * NS
    ZR = 104
    weighted = w is not None

    mesh = plsc.VectorSubcoreMesh(core_axis_name="c", subcore_axis_name="s")
    scratch = [
        [pltpu.VMEM((C,), jnp.int32)] * 2,
        [pltpu.VMEM((C,), jnp.int32)] * 2,
        [pltpu.VMEM((C, Dq), jnp.float32)] * 2,
        pltpu.VMEM((ZR, Dq), jnp.float32),
        pltpu.VMEM_SHARED((N, Dq), jnp.float32),
        [pltpu.SemaphoreType.DMA] * 2,
        [pltpu.SemaphoreType.DMA] * 2,
    ]
    if weighted:
        scratch.append([pltpu.VMEM((C,), jnp.float32)] * 2)

    def body(*refs):
        if weighted:
            (t0, t1, t2, t3, srcr, dstr, wr, o0, o1, o2, o3,
             src_v, dst_v, rows_v, zbuf, acc, sem, sem_sc, w_v) = refs
        else:
            (t0, t1, t2, t3, srcr, dstr, o0, o1, o2, o3,
             src_v, dst_v, rows_v, zbuf, acc, sem, sem_sc) = refs
            w_v = None
        tabs = (t0, t1, t2, t3)
        outs = (o0, o1, o2, o3)
        cid = lax.axis_index("c")
        sid = lax.axis_index("s")
        base_r = sid * RPS
        ebase = sid * e_per

        def zb(r, carry):
            for j in range(Dq // 16):
                zbuf[r, pl.ds(j * 16, 16)] = jnp.zeros((16,), jnp.float32)
            return carry
        lax.fori_loop(0, ZR, zb, 0)

        def load_idx(k, b):
            base = ebase + k * C
            pltpu.sync_copy(srcr.at[pl.ds(base, C)], src_v[b])
            pltpu.sync_copy(dstr.at[pl.ds(base, C)], dst_v[b])
            if weighted:
                pltpu.sync_copy(wr.at[pl.ds(base, C)], w_v[b])

        def start_gather(p, b):
            @pl.when(cid == 0)
            def _():
                pltpu.async_copy(tabs[2 * p].at[src_v[b]], rows_v[b], sem[b])

            @pl.when(cid == 1)
            def _():
                pltpu.async_copy(tabs[2 * p + 1].at[src_v[b]], rows_v[b],
                                 sem[b])

        def wait_gather(p, b):
            @pl.when(cid == 0)
            def _():
                pltpu.make_async_copy(tabs[2 * p].at[src_v[b]], rows_v[b],
                                      sem[b]).wait()

            @pl.when(cid == 1)
            def _():
                pltpu.make_async_copy(tabs[2 * p + 1].at[src_v[b]], rows_v[b],
                                      sem[b]).wait()

        for p in range(2):
            for s in range(RPS // ZR):
                pltpu.sync_copy(zbuf, acc.at[pl.ds(base_r + s * ZR, ZR)])

            @pl.when(sid == NS - 1)
            def _():
                pltpu.sync_copy(zbuf.at[pl.ds(0, TAIL)],
                                acc.at[pl.ds(RPS * NS, TAIL)])
            plsc.subcore_barrier()

            load_idx(0, 0)
            start_gather(p, 0)

            def chunk2(k2, carry):
                for par in (0, 1):
                    k = 2 * k2 + par
                    nb = 1 - par

                    wait_gather(p, par)

                    if weighted:
                        def scale(e2, c2):
                            ws = plsc.load_gather(
                                w_v[par], [jnp.full((16,), e2, jnp.int32)])
                            if invert:
                                ws = 1.0 - ws
                            for j in range(Dq // 16):
                                sl = pl.ds(j * 16, 16)
                                rows_v[par][e2, sl] = rows_v[par][e2, sl] * ws
                            return c2
                        lax.fori_loop(0, C, scale, 0, unroll=8)

                    @pl.when(k >= 1)
                    def _():
                        pltpu.make_async_copy(
                            rows_v[nb], acc.at[dst_v[nb]], sem_sc[nb]).wait()

                    @pl.when(k + 1 < nchunk)
                    def _():
                        load_idx(k + 1, nb)
                        start_gather(p, nb)

                    pltpu.async_copy(rows_v[par], acc.at[dst_v[par]],
                                     sem_sc[par], add=True)
                return carry
            lax.fori_loop(0, nchunk // 2, chunk2, 0)
            pltpu.make_async_copy(rows_v[(nchunk - 1) % 2],
                                  acc.at[dst_v[(nchunk - 1) % 2]],
                                  sem_sc[(nchunk - 1) % 2]).wait()
            plsc.subcore_barrier()

            for q in range(2):
                @pl.when(cid == q)
                def _(q=q):
                    oq = outs[2 * p + q]
                    pltpu.sync_copy(acc.at[pl.ds(base_r, RPS)],
                                    oq.at[pl.ds(base_r, RPS)])

                    @pl.when(sid == NS - 1)
                    def _():
                        pltpu.sync_copy(acc.at[pl.ds(RPS * NS, TAIL)],
                                        oq.at[pl.ds(RPS * NS, TAIL)])
            if p == 0:
                plsc.subcore_barrier()

    out_type = [jax.ShapeDtypeStruct((N, Dq), jnp.float32)] * 4
    k = pl.kernel(body, mesh=mesh, out_type=out_type, scratch_types=scratch,
                  compiler_params=_SC_PARAMS)
    args = list(ts) + [src, dst]
    if weighted:
        args.append(w)
    return k(*args)


def _edge_head_sc(a_s, a_d, src, dst):
    CE = 2000
    e_per = E // (2 * NS)
    nchunk = e_per // CE
    mesh = plsc.VectorSubcoreMesh(core_axis_name="c", subcore_axis_name="s")
    scratch = [
        pltpu.VMEM((N,), jnp.float32),
        pltpu.VMEM((N,), jnp.float32),
        pltpu.VMEM((CE,), jnp.int32),
        pltpu.VMEM((CE,), jnp.int32),
        pltpu.VMEM((CE,), jnp.float32),
        pltpu.VMEM((16,), jnp.float32),
    ]

    def body(asr, adr, srcr, dstr, ecr, psumr, pcntr,
             as_v, ad_v, src_v, dst_v, out_v, sbuf):
        cid = lax.axis_index("c")
        sid = lax.axis_index("s")
        wid = cid * NS + sid
        pltpu.sync_copy(asr, as_v)
        pltpu.sync_copy(adr, ad_v)

        def chunk(k, carry):
            s_, c_ = carry
            base = wid * e_per + k * CE
            pltpu.sync_copy(srcr.at[pl.ds(base, CE)], src_v)
            pltpu.sync_copy(dstr.at[pl.ds(base, CE)], dst_v)

            def inner(i, carry2):
                s2, c2 = carry2
                sl = pl.ds(i * 16, 16)
                va = plsc.load_gather(as_v, [src_v[sl]])
                vd = plsc.load_gather(ad_v, [dst_v[sl]])
                sg = 1.0 / (1.0 + jnp.exp(-(va + vd)))
                out_v[sl] = sg
                c2 = c2 + jnp.where(sg > 0.5, 1.0, 0.0)
                return (s2 + sg, c2)
            s_, c_ = lax.fori_loop(0, CE // 16, inner, (s_, c_), unroll=4)
            pltpu.sync_copy(out_v, ecr.at[pl.ds(base, CE)])
            return (s_, c_)

        z16 = jnp.zeros((16,), jnp.float32)
        s_, c_ = lax.fori_loop(0, nchunk, chunk, (z16, z16))
        sbuf[...] = s_
        pltpu.sync_copy(sbuf, psumr.at[wid])
        sbuf[...] = c_
        pltpu.sync_copy(sbuf, pcntr.at[wid])

    out_type = [
        jax.ShapeDtypeStruct((E,), jnp.float32),
        jax.ShapeDtypeStruct((2 * NS, 16), jnp.float32),
        jax.ShapeDtypeStruct((2 * NS, 16), jnp.float32),
    ]
    k = pl.kernel(body, mesh=mesh, out_type=out_type, scratch_types=scratch,
                  compiler_params=_SC_PARAMS)
    return k(a_s, a_d, src, dst)


BN = 2000
_dot = functools.partial(jnp.dot, preferred_element_type=jnp.float32)


def _q_matmul(zq, wref, bref, qo):
    acc = bref[qo]
    for qi in range(len(zq)):
        acc = acc + _dot(zq[qi], wref[qi, qo])
    return acc


def _first_layers(ax, xs, wf, bf, wc, bc):
    def body(*refs):
        (a0, a1, a2, a3, x0, x1, x2, x3, wfr, bfr, wcr, bcr) = refs[:12]
        houts = refs[12:16]
        couts = refs[16:20]
        zq = [a[...] + x[...] for a, x in
              zip((a0, a1, a2, a3), (x0, x1, x2, x3))]
        for qo in range(4):
            houts[qo][...] = jax.nn.relu(_q_matmul(zq, wfr, bfr, qo))
            couts[qo][...] = jax.nn.relu(_q_matmul(zq, wcr, bcr, qo))

    io = lambda i: (i, 0)
    return pl.pallas_call(
        body, grid=(N // BN,),
        in_specs=[pl.BlockSpec((BN, 32), io)] * 8
        + [pl.BlockSpec((4, 4, 32, Q), lambda i: (0, 0, 0, 0)),
           pl.BlockSpec((4, 1, Q), lambda i: (0, 0, 0)),
           pl.BlockSpec((4, 4, 32, Q), lambda i: (0, 0, 0, 0)),
           pl.BlockSpec((4, 1, Q), lambda i: (0, 0, 0))],
        out_specs=[pl.BlockSpec((BN, Q), io)] * 8,
        out_shape=[jax.ShapeDtypeStruct((N, Q), jnp.float32)] * 8,
    )(*ax, *xs, wf, bf, wc, bc)


def _mid_layer(ag, hs, wp, bp, wh=None, bh=None):
    with_heads = wh is not None

    def body(*refs):
        ins = refs[:8]
        if with_heads:
            wpr, bpr, whr, bhr = refs[8:12]
            outs = refs[12:16]
            hd = refs[16]
        else:
            wpr, bpr = refs[8:10]
            outs = refs[10:14]
        zq = [a[...] + h[...] for a, h in zip(ins[:4], ins[4:])]
        oq = [jax.nn.relu(_q_matmul(zq, wpr, bpr, qo)) for qo in range(4)]
        for qo in range(4):
            outs[qo][...] = oq[qo]
        if with_heads:
            raw = bhr[0]
            for qi in range(4):
                raw = raw + _dot(oq[qi], whr[qi])
            jj = lax.broadcasted_iota(jnp.int32, raw.shape, 1)
            hd[...] = jnp.where(jj == 0, jax.nn.sigmoid(raw), raw)

    io = lambda i: (i, 0)
    in_specs = [pl.BlockSpec((BN, Q), io)] * 8 + [
        pl.BlockSpec((4, 4, Q, Q), lambda i: (0, 0, 0, 0)),
        pl.BlockSpec((4, 1, Q), lambda i: (0, 0, 0)),
    ]
    out_specs = [pl.BlockSpec((BN, Q), io)] * 4
    out_shape = [jax.ShapeDtypeStruct((N, Q), jnp.float32)] * 4
    args = list(ag) + list(hs) + [wp, bp]
    if with_heads:
        in_specs += [pl.BlockSpec((4, Q, 8), lambda i: (0, 0, 0)),
                     pl.BlockSpec((1, 8), lambda i: (0, 0))]
        out_specs += [pl.BlockSpec((BN, 8), io)]
        out_shape += [jax.ShapeDtypeStruct((N, 8), jnp.float32)]
        args += [wh, bh]
    return pl.pallas_call(
        body, grid=(N // BN,), in_specs=in_specs, out_specs=out_specs,
        out_shape=out_shape,
    )(*args)


def _masked_layers(aC, aE, hC, hE, nc, wp, bp, env_diff=False):
    def body(*refs):
        acr = refs[0:4]
        aer = refs[4:8]
        hcr = refs[8:12]
        her = refs[12:16]
        ncr, wpr, bpr = refs[16:19]
        ocr = refs[19:23]
        oer = refs[23:27]
        m = ncr[...]
        zc = [a[...] + h[...] * m for a, h in zip(acr, hcr)]
        if env_diff:
            ze = [(u[...] - c[...]) + h[...] * (1.0 - m)
                  for u, c, h in zip(aer, acr, her)]
        else:
            ze = [a[...] + h[...] * (1.0 - m) for a, h in zip(aer, her)]
        for qo in range(4):
            ocr[qo][...] = jax.nn.relu(_q_matmul(zc, wpr, bpr, qo))
            oer[qo][...] = jax.nn.relu(_q_matmul(ze, wpr, bpr, qo))

    io = lambda i: (i, 0)
    return pl.pallas_call(
        body, grid=(N // BN,),
        in_specs=[pl.BlockSpec((BN, Q), io)] * 16
        + [pl.BlockSpec((BN, 1), io),
           pl.BlockSpec((4, 4, Q, Q), lambda i: (0, 0, 0, 0)),
           pl.BlockSpec((4, 1, Q), lambda i: (0, 0, 0))],
        out_specs=[pl.BlockSpec((BN, Q), io)] * 8,
        out_shape=[jax.ShapeDtypeStruct((N, Q), jnp.float32)] * 8,
    )(*aC, *aE, *hC, *hE, nc, wp, bp)


def _pool_predict(hc, he, batch2, nc, wpred, bpred):
    nsteps = N // BN

    def body(*refs):
        cin = refs[0:4]
        ein = refs[4:8]
        br, ncr, wpr, bpr = refs[8:12]
        accC = refs[12:16]
        accE = refs[16:20]
        cnt, nsum, pC, pE, pA = refs[20:25]
        pid = pl.program_id(0)
        oh = (br[...] == lax.broadcasted_iota(jnp.int32, (BN, G), 1)
              ).astype(jnp.float32)
        dT = lambda a, b: lax.dot_general(
            a, b, (((0,), (0,)), ((), ())),
            preferred_element_type=jnp.float32)
        ncv = ncr[...]
        s0 = jnp.sum(ncv)
        s1 = jnp.sum(jnp.where(ncv > 0.5, 1.0, 0.0))
        ii = lax.broadcasted_iota(jnp.int32, (8, G), 0)
        jj = lax.broadcasted_iota(jnp.int32, (8, G), 1)
        nsv = (jnp.where((ii == 0) & (jj == 0), s0, 0.0)
               + jnp.where((ii == 0) & (jj == 1), s1, 0.0))
        ones = jnp.ones((BN, 8), jnp.float32)

        @pl.when(pid == 0)
        def _():
            for q in range(4):
                accC[q][...] = dT(oh, cin[q][...])
                accE[q][...] = dT(oh, ein[q][...])
            cnt[...] = dT(oh, ones)
            nsum[...] = nsv

        @pl.when(pid != 0)
        def _():
            for q in range(4):
                accC[q][...] += dT(oh, cin[q][...])
                accE[q][...] += dT(oh, ein[q][...])
            cnt[...] += dT(oh, ones)
            nsum[...] += nsv

        @pl.when(pid == nsteps - 1)
        def _():
            denom = jnp.maximum(cnt[...][:, 0:1], 1.0)
            vC = bpr[...]
            vE = bpr[...]
            vA = bpr[...]
            for q in range(4):
                gc = accC[q][...] / denom
                ge = accE[q][...] / denom
                vC = vC + _dot(gc, wpr[q])
                vE = vE + _dot(ge, wpr[q])
                vA = vA + _dot(gc + ge, wpr[q])
            pC[...] = vC
            pE[...] = vE
            pA[...] = vA

    io = lambda i: (i, 0)
    fix = lambda i: (0, 0)
    return pl.pallas_call(
        body, grid=(nsteps,),
        in_specs=[pl.BlockSpec((BN, Q), io)] * 8
        + [pl.BlockSpec((BN, 1), io), pl.BlockSpec((BN, 1), io),
           pl.BlockSpec((4, Q, G), lambda i: (0, 0, 0)),
           pl.BlockSpec((1, G), fix)],
        out_specs=[pl.BlockSpec((G, Q), fix)] * 8
        + [pl.BlockSpec((G, 8), fix), pl.BlockSpec((8, G), fix)]
        + [pl.BlockSpec((G, G), fix)] * 3,
        out_shape=[jax.ShapeDtypeStruct((G, Q), jnp.float32)] * 8
        + [jax.ShapeDtypeStruct((G, 8), jnp.float32),
           jax.ShapeDtypeStruct((8, G), jnp.float32)]
        + [jax.ShapeDtypeStruct((G, G), jnp.float32)] * 3,
    )(*hc, *he, batch2, nc, wpred, bpred)


def _pad_cols_q(W):
    return [jnp.pad(W[:, 75 * q:75 * (q + 1)], ((0, 0), (0, 5)))
            for q in range(4)]


def _pad_rows_q(M):
    return [jnp.pad(M[75 * q:75 * (q + 1)], ((0, 5), (0, 0)))
            for q in range(4)]


def _split_b_q(b):
    return jnp.stack([jnp.pad(b[75 * q:75 * (q + 1)], (0, 5))[None, :]
                      for q in range(4)])


def _mid_w_q(W):
    cols = _pad_cols_q(W)
    return jnp.stack([jnp.stack(_pad_rows_q(c)) for c in cols], axis=1)


def _first_w_q(W):
    cols = _pad_cols_q(W)
    return jnp.stack(
        [jnp.stack([c[32 * qi:32 * (qi + 1)] for qi in range(4)])
         for c in cols], axis=1)


def kernel(x, edge_index, batch,
           W_f0, b_f0, W_f1, b_f1,
           W_c0, b_c0, W_c1, b_c1,
           W_node, b_node, W_edge, b_edge,
           W_b0, b_b0, W_b1, b_b1,
           W_pred, b_pred):
    src = edge_index[0]
    dst = edge_index[1]
    xs = [x[:, 32 * q:32 * (q + 1)] for q in range(4)]

    wf0 = _first_w_q(W_f0)
    wc0 = _first_w_q(W_c0)
    bf0 = _split_b_q(b_f0)
    bc0 = _split_b_q(b_c0)
    wf1 = _mid_w_q(W_f1)
    bf1 = _split_b_q(b_f1)
    wc1 = _mid_w_q(W_c1)
    bc1 = _split_b_q(b_c1)
    wb0 = _mid_w_q(W_b0)
    bb0 = _split_b_q(b_b0)
    wb1 = _mid_w_q(W_b1)
    bb1 = _split_b_q(b_b1)
    wh = jnp.concatenate(
        [W_node, W_edge[:300], W_edge[300:], jnp.zeros((300, 5), jnp.float32)],
        axis=1)
    wh = jnp.stack(_pad_rows_q(wh))
    bh = jnp.array([[b_node[0], 0.0, b_edge[0], 0.0, 0.0, 0.0, 0.0, 0.0]],
                   jnp.float32)
    wpred = jnp.stack(_pad_rows_q(jnp.pad(W_pred, ((0, 0), (0, G - 10)))))
    bpred = jnp.pad(b_pred, (0, G - 10))[None, :]

    ax = _segsum_sc(xs, src, dst, Dq=32, C=1000)
    f0c0 = _first_layers(ax, xs, wf0, bf0, wc0, bc0)
    h, c0 = f0c0[:4], f0c0[4:]

    ah = _segsum_sc(h, src, dst)
    xe = _mid_layer(ah, h, wf1, bf1)

    ac = _segsum_sc(c0, src, dst)
    c1_out = _mid_layer(ac, c0, wc1, bc1, wh, bh)
    heads = c1_out[4]
    nc = heads[:, 0:1]
    a_s = heads[:, 1]
    a_d = heads[:, 2]

    ec, psum, pcnt = _edge_head_sc(a_s, a_d, src, dst)

    aC = _segsum_sc(xe, src, dst, w=ec)
    aU = _segsum_sc(xe, src, dst)
    b0 = _masked_layers(aC, aU, xe, xe, nc, wb0, bb0, env_diff=True)
    h1c, h1e = b0[:4], b0[4:]

    aC2 = _segsum_sc(h1c, src, dst, w=ec)
    aE2 = _segsum_sc(h1e, src, dst, w=ec, invert=True)
    b1 = _masked_layers(aC2, aE2, h1c, h1e, nc, wb1, bb1)
    h2c, h2e = b1[:4], b1[4:]

    outs = _pool_predict(h2c, h2e, batch[:, None], nc, wpred, bpred)
    nsum, pC, pE, pA = outs[9], outs[10], outs[11], outs[12]

    nk = nsum[0, 0]
    rn = nsum[0, 1] / N
    ek = jnp.sum(psum)
    re = jnp.sum(pcnt) / E
    ne = N - nk
    ee = E - ek
    cau_loss_reg = (jnp.abs(nk / (nk + ne) - CAU_GAMMA) + (rn - CAU_GAMMA)
                    + jnp.abs(ek / (ek + ee) - CAU_GAMMA) + (re - CAU_GAMMA))

    return pC[:, :10], pE[:, :10], pA[:, :10], cau_loss_reg


# speedup vs baseline: 1.1007x; 1.1007x over previous
import functools

import jax
import jax.numpy as jnp
from jax import lax
from jax.experimental import pallas as pl
from jax.experimental.pallas import tpu as pltpu
from jax.experimental.pallas import tpu_sc as plsc

N = 10000
E = 320000
G = 128
Q = 80
NS = 16
CAU_GAMMA = 0.4

_SC_PARAMS = pltpu.CompilerParams(use_tc_tiling_on_sc=False,
                                  needs_layout_passes=False)


def _segsum_sc(ts, src, dst, w=None, invert=False, Dq=Q, C=400):
    e_per = E // NS
    nchunk = e_per // C
    assert nchunk % 2 == 0
    RPS = 624
    TAIL = N - RPS * NS
    ZR = 104
    weighted = w is not None

    mesh = plsc.VectorSubcoreMesh(core_axis_name="c", subcore_axis_name="s")
    scratch = [
        [pltpu.VMEM((C,), jnp.int32)] * 2,
        [pltpu.VMEM((C,), jnp.int32)] * 2,
        [pltpu.VMEM((C, Dq), jnp.float32)] * 2,
        pltpu.VMEM((ZR, Dq), jnp.float32),
        pltpu.VMEM_SHARED((N, Dq), jnp.float32),
        [pltpu.SemaphoreType.DMA] * 2,
    ]
    if weighted:
        scratch.append([pltpu.VMEM((C,), jnp.float32)] * 2)

    def body(*refs):
        if weighted:
            (t0, t1, t2, t3, srcr, dstr, wr, o0, o1, o2, o3,
             src_v, dst_v, rows_v, zbuf, acc, sem, w_v) = refs
        else:
            (t0, t1, t2, t3, srcr, dstr, o0, o1, o2, o3,
             src_v, dst_v, rows_v, zbuf, acc, sem) = refs
            w_v = None
        tabs = (t0, t1, t2, t3)
        outs = (o0, o1, o2, o3)
        cid = lax.axis_index("c")
        sid = lax.axis_index("s")
        base_r = sid * RPS
        ebase = sid * e_per

        def zb(r, carry):
            for j in range(Dq // 16):
                zbuf[r, pl.ds(j * 16, 16)] = jnp.zeros((16,), jnp.float32)
            return carry
        lax.fori_loop(0, ZR, zb, 0)

        def load_idx(k, b):
            base = ebase + k * C
            pltpu.sync_copy(srcr.at[pl.ds(base, C)], src_v[b])
            pltpu.sync_copy(dstr.at[pl.ds(base, C)], dst_v[b])
            if weighted:
                pltpu.sync_copy(wr.at[pl.ds(base, C)], w_v[b])

        def start_gather(p, b):
            @pl.when(cid == 0)
            def _():
                pltpu.async_copy(tabs[2 * p].at[src_v[b]], rows_v[b], sem[b])

            @pl.when(cid == 1)
            def _():
                pltpu.async_copy(tabs[2 * p + 1].at[src_v[b]], rows_v[b],
                                 sem[b])

        def wait_gather(p, b):
            @pl.when(cid == 0)
            def _():
                pltpu.make_async_copy(tabs[2 * p].at[src_v[b]], rows_v[b],
                                      sem[b]).wait()

            @pl.when(cid == 1)
            def _():
                pltpu.make_async_copy(tabs[2 * p + 1].at[src_v[b]], rows_v[b],
                                      sem[b]).wait()

        for p in range(2):
            for s in range(RPS // ZR):
                pltpu.sync_copy(zbuf, acc.at[pl.ds(base_r + s * ZR, ZR)])

            @pl.when(sid == NS - 1)
            def _():
                pltpu.sync_copy(zbuf.at[pl.ds(0, TAIL)],
                                acc.at[pl.ds(RPS * NS, TAIL)])
            plsc.subcore_barrier()

            load_idx(0, 0)
            start_gather(p, 0)

            def chunk2(k2, carry):
                for par in (0, 1):
                    k = 2 * k2 + par
                    nb = 1 - par

                    @pl.when(k + 1 < nchunk)
                    def _():
                        load_idx(k + 1, nb)
                        start_gather(p, nb)

                    wait_gather(p, par)

                    if weighted:
                        def scale(g, c2):
                            w16 = w_v[par][pl.ds(g * 16, 16)]
                            if invert:
                                w16 = 1.0 - w16
                            for e16 in range(16):
                                ws = jnp.full((16,), w16[e16])
                                e2 = g * 16 + e16
                                for j in range(Dq // 16):
                                    sl = pl.ds(j * 16, 16)
                                    rows_v[par][e2, sl] = (
                                        rows_v[par][e2, sl] * ws)
                            return c2
                        lax.fori_loop(0, C // 16, scale, 0)

                    pltpu.sync_copy(rows_v[par], acc.at[dst_v[par]], add=True)
                return carry
            lax.fori_loop(0, nchunk // 2, chunk2, 0)
            plsc.subcore_barrier()

            for q in range(2):
                @pl.when(cid == q)
                def _(q=q):
                    oq = outs[2 * p + q]
                    pltpu.sync_copy(acc.at[pl.ds(base_r, RPS)],
                                    oq.at[pl.ds(base_r, RPS)])

                    @pl.when(sid == NS - 1)
                    def _():
                        pltpu.sync_copy(acc.at[pl.ds(RPS * NS, TAIL)],
                                        oq.at[pl.ds(RPS * NS, TAIL)])
            if p == 0:
                plsc.subcore_barrier()

    out_type = [jax.ShapeDtypeStruct((N, Dq), jnp.float32)] * 4
    k = pl.kernel(body, mesh=mesh, out_type=out_type, scratch_types=scratch,
                  compiler_params=_SC_PARAMS)
    args = list(ts) + [src, dst]
    if weighted:
        args.append(w)
    return k(*args)


def _edge_head_sc(a_s, a_d, src, dst):
    CE = 2000
    e_per = E // (2 * NS)
    nchunk = e_per // CE
    mesh = plsc.VectorSubcoreMesh(core_axis_name="c", subcore_axis_name="s")
    scratch = [
        pltpu.VMEM((N,), jnp.float32),
        pltpu.VMEM((N,), jnp.float32),
        pltpu.VMEM((CE,), jnp.int32),
        pltpu.VMEM((CE,), jnp.int32),
        pltpu.VMEM((CE,), jnp.float32),
        pltpu.VMEM((16,), jnp.float32),
    ]

    def body(asr, adr, srcr, dstr, ecr, psumr, pcntr,
             as_v, ad_v, src_v, dst_v, out_v, sbuf):
        cid = lax.axis_index("c")
        sid = lax.axis_index("s")
        wid = cid * NS + sid
        pltpu.sync_copy(asr, as_v)
        pltpu.sync_copy(adr, ad_v)

        def chunk(k, carry):
            s_, c_ = carry
            base = wid * e_per + k * CE
            pltpu.sync_copy(srcr.at[pl.ds(base, CE)], src_v)
            pltpu.sync_copy(dstr.at[pl.ds(base, CE)], dst_v)

            def inner(i, carry2):
                s2, c2 = carry2
                sl = pl.ds(i * 16, 16)
                va = plsc.load_gather(as_v, [src_v[sl]])
                vd = plsc.load_gather(ad_v, [dst_v[sl]])
                sg = 1.0 / (1.0 + jnp.exp(-(va + vd)))
                out_v[sl] = sg
                c2 = c2 + jnp.where(sg > 0.5, 1.0, 0.0)
                return (s2 + sg, c2)
            s_, c_ = lax.fori_loop(0, CE // 16, inner, (s_, c_), unroll=4)
            pltpu.sync_copy(out_v, ecr.at[pl.ds(base, CE)])
            return (s_, c_)

        z16 = jnp.zeros((16,), jnp.float32)
        s_, c_ = lax.fori_loop(0, nchunk, chunk, (z16, z16))
        sbuf[...] = s_
        pltpu.sync_copy(sbuf, psumr.at[wid])
        sbuf[...] = c_
        pltpu.sync_copy(sbuf, pcntr.at[wid])

    out_type = [
        jax.ShapeDtypeStruct((E,), jnp.float32),
        jax.ShapeDtypeStruct((2 * NS, 16), jnp.float32),
        jax.ShapeDtypeStruct((2 * NS, 16), jnp.float32),
    ]
    k = pl.kernel(body, mesh=mesh, out_type=out_type, scratch_types=scratch,
                  compiler_params=_SC_PARAMS)
    return k(a_s, a_d, src, dst)


BN = 2000
_dot = functools.partial(jnp.dot, preferred_element_type=jnp.float32)


def _q_matmul(zq, wref, bref, qo):
    acc = bref[qo]
    for qi in range(len(zq)):
        acc = acc + _dot(zq[qi], wref[qi, qo])
    return acc


def _first_layers(ax, xs, wf, bf, wc, bc):
    def body(*refs):
        (a0, a1, a2, a3, x0, x1, x2, x3, wfr, bfr, wcr, bcr) = refs[:12]
        houts = refs[12:16]
        couts = refs[16:20]
        zq = [a[...] + x[...] for a, x in
              zip((a0, a1, a2, a3), (x0, x1, x2, x3))]
        for qo in range(4):
            houts[qo][...] = jax.nn.relu(_q_matmul(zq, wfr, bfr, qo))
            couts[qo][...] = jax.nn.relu(_q_matmul(zq, wcr, bcr, qo))

    io = lambda i: (i, 0)
    return pl.pallas_call(
        body, grid=(N // BN,),
        in_specs=[pl.BlockSpec((BN, 32), io)] * 8
        + [pl.BlockSpec((4, 4, 32, Q), lambda i: (0, 0, 0, 0)),
           pl.BlockSpec((4, 1, Q), lambda i: (0, 0, 0)),
           pl.BlockSpec((4, 4, 32, Q), lambda i: (0, 0, 0, 0)),
           pl.BlockSpec((4, 1, Q), lambda i: (0, 0, 0))],
        out_specs=[pl.BlockSpec((BN, Q), io)] * 8,
        out_shape=[jax.ShapeDtypeStruct((N, Q), jnp.float32)] * 8,
    )(*ax, *xs, wf, bf, wc, bc)


def _mid_layer(ag, hs, wp, bp, wh=None, bh=None):
    with_heads = wh is not None

    def body(*refs):
        ins = refs[:8]
        if with_heads:
            wpr, bpr, whr, bhr = refs[8:12]
            outs = refs[12:16]
            hd = refs[16]
        else:
            wpr, bpr = refs[8:10]
            outs = refs[10:14]
        zq = [a[...] + h[...] for a, h in zip(ins[:4], ins[4:])]
        oq = [jax.nn.relu(_q_matmul(zq, wpr, bpr, qo)) for qo in range(4)]
        for qo in range(4):
            outs[qo][...] = oq[qo]
        if with_heads:
            raw = bhr[0]
            for qi in range(4):
                raw = raw + _dot(oq[qi], whr[qi])
            jj = lax.broadcasted_iota(jnp.int32, raw.shape, 1)
            hd[...] = jnp.where(jj == 0, jax.nn.sigmoid(raw), raw)

    io = lambda i: (i, 0)
    in_specs = [pl.BlockSpec((BN, Q), io)] * 8 + [
        pl.BlockSpec((4, 4, Q, Q), lambda i: (0, 0, 0, 0)),
        pl.BlockSpec((4, 1, Q), lambda i: (0, 0, 0)),
    ]
    out_specs = [pl.BlockSpec((BN, Q), io)] * 4
    out_shape = [jax.ShapeDtypeStruct((N, Q), jnp.float32)] * 4
    args = list(ag) + list(hs) + [wp, bp]
    if with_heads:
        in_specs += [pl.BlockSpec((4, Q, 8), lambda i: (0, 0, 0)),
                     pl.BlockSpec((1, 8), lambda i: (0, 0))]
        out_specs += [pl.BlockSpec((BN, 8), io)]
        out_shape += [jax.ShapeDtypeStruct((N, 8), jnp.float32)]
        args += [wh, bh]
    return pl.pallas_call(
        body, grid=(N // BN,), in_specs=in_specs, out_specs=out_specs,
        out_shape=out_shape,
    )(*args)


def _masked_layers(aC, aE, hC, hE, nc, wp, bp, env_diff=False):
    def body(*refs):
        acr = refs[0:4]
        aer = refs[4:8]
        hcr = refs[8:12]
        her = refs[12:16]
        ncr, wpr, bpr = refs[16:19]
        ocr = refs[19:23]
        oer = refs[23:27]
        m = ncr[...]
        zc = [a[...] + h[...] * m for a, h in zip(acr, hcr)]
        if env_diff:
            ze = [(u[...] - c[...]) + h[...] * (1.0 - m)
                  for u, c, h in zip(aer, acr, her)]
        else:
            ze = [a[...] + h[...] * (1.0 - m) for a, h in zip(aer, her)]
        for qo in range(4):
            ocr[qo][...] = jax.nn.relu(_q_matmul(zc, wpr, bpr, qo))
            oer[qo][...] = jax.nn.relu(_q_matmul(ze, wpr, bpr, qo))

    io = lambda i: (i, 0)
    return pl.pallas_call(
        body, grid=(N // BN,),
        in_specs=[pl.BlockSpec((BN, Q), io)] * 16
        + [pl.BlockSpec((BN, 1), io),
           pl.BlockSpec((4, 4, Q, Q), lambda i: (0, 0, 0, 0)),
           pl.BlockSpec((4, 1, Q), lambda i: (0, 0, 0))],
        out_specs=[pl.BlockSpec((BN, Q), io)] * 8,
        out_shape=[jax.ShapeDtypeStruct((N, Q), jnp.float32)] * 8,
    )(*aC, *aE, *hC, *hE, nc, wp, bp)


def _pool_predict(hc, he, batch2, nc, wpred, bpred):
    nsteps = N // BN

    def body(*refs):
        cin = refs[0:4]
        ein = refs[4:8]
        br, ncr, wpr, bpr = refs[8:12]
        accC = refs[12:16]
        accE = refs[16:20]
        cnt, nsum, pC, pE, pA = refs[20:25]
        pid = pl.program_id(0)
        oh = (br[...] == lax.broadcasted_iota(jnp.int32, (BN, G), 1)
              ).astype(jnp.float32)
        dT = lambda a, b: lax.dot_general(
            a, b, (((0,), (0,)), ((), ())),
            preferred_element_type=jnp.float32)
        ncv = ncr[...]
        s0 = jnp.sum(ncv)
        s1 = jnp.sum(jnp.where(ncv > 0.5, 1.0, 0.0))
        ii = lax.broadcasted_iota(jnp.int32, (8, G), 0)
        jj = lax.broadcasted_iota(jnp.int32, (8, G), 1)
        nsv = (jnp.where((ii == 0) & (jj == 0), s0, 0.0)
               + jnp.where((ii == 0) & (jj == 1), s1, 0.0))
        ones = jnp.ones((BN, 8), jnp.float32)

        @pl.when(pid == 0)
        def _():
            for q in range(4):
                accC[q][...] = dT(oh, cin[q][...])
                accE[q][...] = dT(oh, ein[q][...])
            cnt[...] = dT(oh, ones)
            nsum[...] = nsv

        @pl.when(pid != 0)
        def _():
            for q in range(4):
                accC[q][...] += dT(oh, cin[q][...])
                accE[q][...] += dT(oh, ein[q][...])
            cnt[...] += dT(oh, ones)
            nsum[...] += nsv

        @pl.when(pid == nsteps - 1)
        def _():
            denom = jnp.maximum(cnt[...][:, 0:1], 1.0)
            vC = bpr[...]
            vE = bpr[...]
            vA = bpr[...]
            for q in range(4):
                gc = accC[q][...] / denom
                ge = accE[q][...] / denom
                vC = vC + _dot(gc, wpr[q])
                vE = vE + _dot(ge, wpr[q])
                vA = vA + _dot(gc + ge, wpr[q])
            pC[...] = vC
            pE[...] = vE
            pA[...] = vA

    io = lambda i: (i, 0)
    fix = lambda i: (0, 0)
    return pl.pallas_call(
        body, grid=(nsteps,),
        in_specs=[pl.BlockSpec((BN, Q), io)] * 8
        + [pl.BlockSpec((BN, 1), io), pl.BlockSpec((BN, 1), io),
           pl.BlockSpec((4, Q, G), lambda i: (0, 0, 0)),
           pl.BlockSpec((1, G), fix)],
        out_specs=[pl.BlockSpec((G, Q), fix)] * 8
        + [pl.BlockSpec((G, 8), fix), pl.BlockSpec((8, G), fix)]
        + [pl.BlockSpec((G, G), fix)] * 3,
        out_shape=[jax.ShapeDtypeStruct((G, Q), jnp.float32)] * 8
        + [jax.ShapeDtypeStruct((G, 8), jnp.float32),
           jax.ShapeDtypeStruct((8, G), jnp.float32)]
        + [jax.ShapeDtypeStruct((G, G), jnp.float32)] * 3,
    )(*hc, *he, batch2, nc, wpred, bpred)


def _pad_cols_q(W):
    return [jnp.pad(W[:, 75 * q:75 * (q + 1)], ((0, 0), (0, 5)))
            for q in range(4)]


def _pad_rows_q(M):
    return [jnp.pad(M[75 * q:75 * (q + 1)], ((0, 5), (0, 0)))
            for q in range(4)]


def _split_b_q(b):
    return jnp.stack([jnp.pad(b[75 * q:75 * (q + 1)], (0, 5))[None, :]
                      for q in range(4)])


def _mid_w_q(W):
    cols = _pad_cols_q(W)
    return jnp.stack([jnp.stack(_pad_rows_q(c)) for c in cols], axis=1)


def _first_w_q(W):
    cols = _pad_cols_q(W)
    return jnp.stack(
        [jnp.stack([c[32 * qi:32 * (qi + 1)] for qi in range(4)])
         for c in cols], axis=1)


def kernel(x, edge_index, batch,
           W_f0, b_f0, W_f1, b_f1,
           W_c0, b_c0, W_c1, b_c1,
           W_node, b_node, W_edge, b_edge,
           W_b0, b_b0, W_b1, b_b1,
           W_pred, b_pred):
    src = edge_index[0]
    dst = edge_index[1]
    xs = [x[:, 32 * q:32 * (q + 1)] for q in range(4)]

    wf0 = _first_w_q(W_f0)
    wc0 = _first_w_q(W_c0)
    bf0 = _split_b_q(b_f0)
    bc0 = _split_b_q(b_c0)
    wf1 = _mid_w_q(W_f1)
    bf1 = _split_b_q(b_f1)
    wc1 = _mid_w_q(W_c1)
    bc1 = _split_b_q(b_c1)
    wb0 = _mid_w_q(W_b0)
    bb0 = _split_b_q(b_b0)
    wb1 = _mid_w_q(W_b1)
    bb1 = _split_b_q(b_b1)
    wh = jnp.concatenate(
        [W_node, W_edge[:300], W_edge[300:], jnp.zeros((300, 5), jnp.float32)],
        axis=1)
    wh = jnp.stack(_pad_rows_q(wh))
    bh = jnp.array([[b_node[0], 0.0, b_edge[0], 0.0, 0.0, 0.0, 0.0, 0.0]],
                   jnp.float32)
    wpred = jnp.stack(_pad_rows_q(jnp.pad(W_pred, ((0, 0), (0, G - 10)))))
    bpred = jnp.pad(b_pred, (0, G - 10))[None, :]

    ax = _segsum_sc(xs, src, dst, Dq=32, C=1000)
    f0c0 = _first_layers(ax, xs, wf0, bf0, wc0, bc0)
    h, c0 = f0c0[:4], f0c0[4:]

    ah = _segsum_sc(h, src, dst)
    xe = _mid_layer(ah, h, wf1, bf1)

    ac = _segsum_sc(c0, src, dst)
    c1_out = _mid_layer(ac, c0, wc1, bc1, wh, bh)
    heads = c1_out[4]
    nc = heads[:, 0:1]
    a_s = heads[:, 1]
    a_d = heads[:, 2]

    ec, psum, pcnt = _edge_head_sc(a_s, a_d, src, dst)

    aC = _segsum_sc(xe, src, dst, w=ec)
    aU = _segsum_sc(xe, src, dst)
    b0 = _masked_layers(aC, aU, xe, xe, nc, wb0, bb0, env_diff=True)
    h1c, h1e = b0[:4], b0[4:]

    aC2 = _segsum_sc(h1c, src, dst, w=ec)
    aE2 = _segsum_sc(h1e, src, dst, w=ec, invert=True)
    b1 = _masked_layers(aC2, aE2, h1c, h1e, nc, wb1, bb1)
    h2c, h2e = b1[:4], b1[4:]

    outs = _pool_predict(h2c, h2e, batch[:, None], nc, wpred, bpred)
    nsum, pC, pE, pA = outs[9], outs[10], outs[11], outs[12]

    nk = nsum[0, 0]
    rn = nsum[0, 1] / N
    ek = jnp.sum(psum)
    re = jnp.sum(pcnt) / E
    ne = N - nk
    ee = E - ek
    cau_loss_reg = (jnp.abs(nk / (nk + ne) - CAU_GAMMA) + (rn - CAU_GAMMA)
                    + jnp.abs(ek / (ek + ee) - CAU_GAMMA) + (re - CAU_GAMMA))

    return pC[:, :10], pE[:, :10], pA[:, :10], cau_loss_reg


# speedup vs baseline: 1.1621x; 1.0558x over previous
import functools

import jax
import jax.numpy as jnp
from jax import lax
from jax.experimental import pallas as pl
from jax.experimental.pallas import tpu as pltpu
from jax.experimental.pallas import tpu_sc as plsc

N = 10000
E = 320000
G = 128
Q = 80
NS = 16
CAU_GAMMA = 0.4

_SC_PARAMS = pltpu.CompilerParams(use_tc_tiling_on_sc=False,
                                  needs_layout_passes=False)


def _segsum_sc(ts, src, dst, w=None, invert=False, Dq=Q, C=400):
    nt = len(ts)
    e_per = E // NS
    nchunk = e_per // C
    assert nchunk % 2 == 0
    RPS = 624
    TAIL = N - RPS * NS
    ZR = 104
    weighted = w is not None

    mesh = plsc.VectorSubcoreMesh(core_axis_name="c", subcore_axis_name="s")
    scratch = [
        [pltpu.VMEM((C,), jnp.int32)] * 2,
        [pltpu.VMEM((C,), jnp.int32)] * 2,
        [pltpu.VMEM((C, Dq), jnp.float32)] * 2,
        pltpu.VMEM((ZR, Dq), jnp.float32),
        pltpu.VMEM_SHARED((N, Dq), jnp.float32),
        [pltpu.SemaphoreType.DMA] * 2,
    ]
    if weighted:
        scratch.append([pltpu.VMEM((C,), jnp.float32)] * 2)

    def body(*refs):
        tabs = refs[:nt]
        srcr, dstr = refs[nt], refs[nt + 1]
        i = nt + 2
        if weighted:
            wr = refs[i]
            i += 1
        outs = refs[i:i + nt]
        i += nt
        src_v, dst_v, rows_v, zbuf, acc, sem = refs[i:i + 6]
        w_v = refs[i + 6] if weighted else None
        cid = lax.axis_index("c")
        sid = lax.axis_index("s")
        base_r = sid * RPS
        ebase = sid * e_per

        def zb(r, carry):
            for j in range(Dq // 16):
                zbuf[r, pl.ds(j * 16, 16)] = jnp.zeros((16,), jnp.float32)
            return carry
        lax.fori_loop(0, ZR, zb, 0)

        def load_idx(k, b):
            base = ebase + k * C
            pltpu.sync_copy(srcr.at[pl.ds(base, C)], src_v[b])
            pltpu.sync_copy(dstr.at[pl.ds(base, C)], dst_v[b])
            if weighted:
                pltpu.sync_copy(wr.at[pl.ds(base, C)], w_v[b])

        def start_gather(p, b):
            @pl.when(cid == 0)
            def _():
                pltpu.async_copy(tabs[2 * p].at[src_v[b]], rows_v[b], sem[b])

            @pl.when(cid == 1)
            def _():
                pltpu.async_copy(tabs[2 * p + 1].at[src_v[b]], rows_v[b],
                                 sem[b])

        def wait_gather(p, b):
            @pl.when(cid == 0)
            def _():
                pltpu.make_async_copy(tabs[2 * p].at[src_v[b]], rows_v[b],
                                      sem[b]).wait()

            @pl.when(cid == 1)
            def _():
                pltpu.make_async_copy(tabs[2 * p + 1].at[src_v[b]], rows_v[b],
                                      sem[b]).wait()

        for p in range(nt // 2):
            for s in range(RPS // ZR):
                pltpu.sync_copy(zbuf, acc.at[pl.ds(base_r + s * ZR, ZR)])

            @pl.when(sid == NS - 1)
            def _():
                pltpu.sync_copy(zbuf.at[pl.ds(0, TAIL)],
                                acc.at[pl.ds(RPS * NS, TAIL)])
            plsc.subcore_barrier()

            load_idx(0, 0)
            start_gather(p, 0)

            def chunk2(k2, carry):
                for par in (0, 1):
                    k = 2 * k2 + par
                    nb = 1 - par

                    @pl.when(k + 1 < nchunk)
                    def _():
                        load_idx(k + 1, nb)
                        start_gather(p, nb)

                    wait_gather(p, par)

                    if weighted:
                        def scale(e2, c2):
                            ws = plsc.load_gather(
                                w_v[par], [jnp.full((16,), e2, jnp.int32)])
                            if invert:
                                ws = 1.0 - ws
                            for j in range(Dq // 16):
                                sl = pl.ds(j * 16, 16)
                                rows_v[par][e2, sl] = rows_v[par][e2, sl] * ws
                            return c2
                        lax.fori_loop(0, C, scale, 0, unroll=8)

                    pltpu.sync_copy(rows_v[par], acc.at[dst_v[par]], add=True)
                return carry
            lax.fori_loop(0, nchunk // 2, chunk2, 0)
            plsc.subcore_barrier()

            for q in range(2):
                @pl.when(cid == q)
                def _(q=q):
                    oq = outs[2 * p + q]
                    pltpu.sync_copy(acc.at[pl.ds(base_r, RPS)],
                                    oq.at[pl.ds(base_r, RPS)])

                    @pl.when(sid == NS - 1)
                    def _():
                        pltpu.sync_copy(acc.at[pl.ds(RPS * NS, TAIL)],
                                        oq.at[pl.ds(RPS * NS, TAIL)])
            if p < nt // 2 - 1:
                plsc.subcore_barrier()

    out_type = [jax.ShapeDtypeStruct((N, Dq), jnp.float32)] * nt
    k = pl.kernel(body, mesh=mesh, out_type=out_type, scratch_types=scratch,
                  compiler_params=_SC_PARAMS)
    args = list(ts) + [src, dst]
    if weighted:
        args.append(w)
    return k(*args)


def _edge_head_sc(a_s, a_d, src, dst):
    CE = 2000
    e_per = E // (2 * NS)
    nchunk = e_per // CE
    mesh = plsc.VectorSubcoreMesh(core_axis_name="c", subcore_axis_name="s")
    scratch = [
        pltpu.VMEM((N,), jnp.float32),
        pltpu.VMEM((N,), jnp.float32),
        pltpu.VMEM((CE,), jnp.int32),
        pltpu.VMEM((CE,), jnp.int32),
        pltpu.VMEM((CE,), jnp.float32),
        pltpu.VMEM((16,), jnp.float32),
    ]

    def body(asr, adr, srcr, dstr, ecr, psumr, pcntr,
             as_v, ad_v, src_v, dst_v, out_v, sbuf):
        cid = lax.axis_index("c")
        sid = lax.axis_index("s")
        wid = cid * NS + sid
        pltpu.sync_copy(asr, as_v)
        pltpu.sync_copy(adr, ad_v)

        def chunk(k, carry):
            s_, c_ = carry
            base = wid * e_per + k * CE
            pltpu.sync_copy(srcr.at[pl.ds(base, CE)], src_v)
            pltpu.sync_copy(dstr.at[pl.ds(base, CE)], dst_v)

            def inner(i, carry2):
                s2, c2 = carry2
                sl = pl.ds(i * 16, 16)
                va = plsc.load_gather(as_v, [src_v[sl]])
                vd = plsc.load_gather(ad_v, [dst_v[sl]])
                sg = 1.0 / (1.0 + jnp.exp(-(va + vd)))
                out_v[sl] = sg
                c2 = c2 + jnp.where(sg > 0.5, 1.0, 0.0)
                return (s2 + sg, c2)
            s_, c_ = lax.fori_loop(0, CE // 16, inner, (s_, c_), unroll=4)
            pltpu.sync_copy(out_v, ecr.at[pl.ds(base, CE)])
            return (s_, c_)

        z16 = jnp.zeros((16,), jnp.float32)
        s_, c_ = lax.fori_loop(0, nchunk, chunk, (z16, z16))
        sbuf[...] = s_
        pltpu.sync_copy(sbuf, psumr.at[wid])
        sbuf[...] = c_
        pltpu.sync_copy(sbuf, pcntr.at[wid])

    out_type = [
        jax.ShapeDtypeStruct((E,), jnp.float32),
        jax.ShapeDtypeStruct((2 * NS, 16), jnp.float32),
        jax.ShapeDtypeStruct((2 * NS, 16), jnp.float32),
    ]
    k = pl.kernel(body, mesh=mesh, out_type=out_type, scratch_types=scratch,
                  compiler_params=_SC_PARAMS)
    return k(a_s, a_d, src, dst)


BN = 2000
_dot = functools.partial(jnp.dot, preferred_element_type=jnp.float32)


def _q_matmul(zq, wref, bref, qo):
    acc = bref[qo]
    for qi in range(len(zq)):
        acc = acc + _dot(zq[qi], wref[qi, qo])
    return acc


def _first_layers(ax, xs, wf, bf, wc, bc):
    def body(*refs):
        (a0, a1, x0, x1, wfr, bfr, wcr, bcr) = refs[:8]
        houts = refs[8:12]
        couts = refs[12:16]
        zq = [a[...] + x[...] for a, x in zip((a0, a1), (x0, x1))]
        for qo in range(4):
            houts[qo][...] = jax.nn.relu(_q_matmul(zq, wfr, bfr, qo))
            couts[qo][...] = jax.nn.relu(_q_matmul(zq, wcr, bcr, qo))

    io = lambda i: (i, 0)
    return pl.pallas_call(
        body, grid=(N // BN,),
        in_specs=[pl.BlockSpec((BN, 64), io)] * 4
        + [pl.BlockSpec((2, 4, 64, Q), lambda i: (0, 0, 0, 0)),
           pl.BlockSpec((4, 1, Q), lambda i: (0, 0, 0)),
           pl.BlockSpec((2, 4, 64, Q), lambda i: (0, 0, 0, 0)),
           pl.BlockSpec((4, 1, Q), lambda i: (0, 0, 0))],
        out_specs=[pl.BlockSpec((BN, Q), io)] * 8,
        out_shape=[jax.ShapeDtypeStruct((N, Q), jnp.float32)] * 8,
    )(*ax, *xs, wf, bf, wc, bc)


def _mid_layer(ag, hs, wp, bp, wh=None, bh=None):
    with_heads = wh is not None

    def body(*refs):
        ins = refs[:8]
        if with_heads:
            wpr, bpr, whr, bhr = refs[8:12]
            outs = refs[12:16]
            hd = refs[16]
        else:
            wpr, bpr = refs[8:10]
            outs = refs[10:14]
        zq = [a[...] + h[...] for a, h in zip(ins[:4], ins[4:])]
        oq = [jax.nn.relu(_q_matmul(zq, wpr, bpr, qo)) for qo in range(4)]
        for qo in range(4):
            outs[qo][...] = oq[qo]
        if with_heads:
            raw = bhr[0]
            for qi in range(4):
                raw = raw + _dot(oq[qi], whr[qi])
            jj = lax.broadcasted_iota(jnp.int32, raw.shape, 1)
            hd[...] = jnp.where(jj == 0, jax.nn.sigmoid(raw), raw)

    io = lambda i: (i, 0)
    in_specs = [pl.BlockSpec((BN, Q), io)] * 8 + [
        pl.BlockSpec((4, 4, Q, Q), lambda i: (0, 0, 0, 0)),
        pl.BlockSpec((4, 1, Q), lambda i: (0, 0, 0)),
    ]
    out_specs = [pl.BlockSpec((BN, Q), io)] * 4
    out_shape = [jax.ShapeDtypeStruct((N, Q), jnp.float32)] * 4
    args = list(ag) + list(hs) + [wp, bp]
    if with_heads:
        in_specs += [pl.BlockSpec((4, Q, 8), lambda i: (0, 0, 0)),
                     pl.BlockSpec((1, 8), lambda i: (0, 0))]
        out_specs += [pl.BlockSpec((BN, 8), io)]
        out_shape += [jax.ShapeDtypeStruct((N, 8), jnp.float32)]
        args += [wh, bh]
    return pl.pallas_call(
        body, grid=(N // BN,), in_specs=in_specs, out_specs=out_specs,
        out_shape=out_shape,
    )(*args)


def _masked_layers(aC, aE, hC, hE, nc, wp, bp, env_diff=False):
    def body(*refs):
        acr = refs[0:4]
        aer = refs[4:8]
        hcr = refs[8:12]
        her = refs[12:16]
        ncr, wpr, bpr = refs[16:19]
        ocr = refs[19:23]
        oer = refs[23:27]
        m = ncr[...]
        zc = [a[...] + h[...] * m for a, h in zip(acr, hcr)]
        if env_diff:
            ze = [(u[...] - c[...]) + h[...] * (1.0 - m)
                  for u, c, h in zip(aer, acr, her)]
        else:
            ze = [a[...] + h[...] * (1.0 - m) for a, h in zip(aer, her)]
        for qo in range(4):
            ocr[qo][...] = jax.nn.relu(_q_matmul(zc, wpr, bpr, qo))
            oer[qo][...] = jax.nn.relu(_q_matmul(ze, wpr, bpr, qo))

    io = lambda i: (i, 0)
    return pl.pallas_call(
        body, grid=(N // BN,),
        in_specs=[pl.BlockSpec((BN, Q), io)] * 16
        + [pl.BlockSpec((BN, 1), io),
           pl.BlockSpec((4, 4, Q, Q), lambda i: (0, 0, 0, 0)),
           pl.BlockSpec((4, 1, Q), lambda i: (0, 0, 0))],
        out_specs=[pl.BlockSpec((BN, Q), io)] * 8,
        out_shape=[jax.ShapeDtypeStruct((N, Q), jnp.float32)] * 8,
    )(*aC, *aE, *hC, *hE, nc, wp, bp)


def _pool_predict(hc, he, batch2, nc, wpred, bpred):
    nsteps = N // BN

    def body(*refs):
        cin = refs[0:4]
        ein = refs[4:8]
        br, ncr, wpr, bpr = refs[8:12]
        accC = refs[12:16]
        accE = refs[16:20]
        cnt, nsum, pC, pE, pA = refs[20:25]
        pid = pl.program_id(0)
        oh = (br[...] == lax.broadcasted_iota(jnp.int32, (BN, G), 1)
              ).astype(jnp.float32)
        dT = lambda a, b: lax.dot_general(
            a, b, (((0,), (0,)), ((), ())),
            preferred_element_type=jnp.float32)
        ncv = ncr[...]
        s0 = jnp.sum(ncv)
        s1 = jnp.sum(jnp.where(ncv > 0.5, 1.0, 0.0))
        ii = lax.broadcasted_iota(jnp.int32, (8, G), 0)
        jj = lax.broadcasted_iota(jnp.int32, (8, G), 1)
        nsv = (jnp.where((ii == 0) & (jj == 0), s0, 0.0)
               + jnp.where((ii == 0) & (jj == 1), s1, 0.0))
        ones = jnp.ones((BN, 8), jnp.float32)

        @pl.when(pid == 0)
        def _():
            for q in range(4):
                accC[q][...] = dT(oh, cin[q][...])
                accE[q][...] = dT(oh, ein[q][...])
            cnt[...] = dT(oh, ones)
            nsum[...] = nsv

        @pl.when(pid != 0)
        def _():
            for q in range(4):
                accC[q][...] += dT(oh, cin[q][...])
                accE[q][...] += dT(oh, ein[q][...])
            cnt[...] += dT(oh, ones)
            nsum[...] += nsv

        @pl.when(pid == nsteps - 1)
        def _():
            denom = jnp.maximum(cnt[...][:, 0:1], 1.0)
            vC = bpr[...]
            vE = bpr[...]
            vA = bpr[...]
            for q in range(4):
                gc = accC[q][...] / denom
                ge = accE[q][...] / denom
                vC = vC + _dot(gc, wpr[q])
                vE = vE + _dot(ge, wpr[q])
                vA = vA + _dot(gc + ge, wpr[q])
            pC[...] = vC
            pE[...] = vE
            pA[...] = vA

    io = lambda i: (i, 0)
    fix = lambda i: (0, 0)
    return pl.pallas_call(
        body, grid=(nsteps,),
        in_specs=[pl.BlockSpec((BN, Q), io)] * 8
        + [pl.BlockSpec((BN, 1), io), pl.BlockSpec((BN, 1), io),
           pl.BlockSpec((4, Q, G), lambda i: (0, 0, 0)),
           pl.BlockSpec((1, G), fix)],
        out_specs=[pl.BlockSpec((G, Q), fix)] * 8
        + [pl.BlockSpec((G, 8), fix), pl.BlockSpec((8, G), fix)]
        + [pl.BlockSpec((G, G), fix)] * 3,
        out_shape=[jax.ShapeDtypeStruct((G, Q), jnp.float32)] * 8
        + [jax.ShapeDtypeStruct((G, 8), jnp.float32),
           jax.ShapeDtypeStruct((8, G), jnp.float32)]
        + [jax.ShapeDtypeStruct((G, G), jnp.float32)] * 3,
    )(*hc, *he, batch2, nc, wpred, bpred)


def _pad_cols_q(W):
    return [jnp.pad(W[:, 75 * q:75 * (q + 1)], ((0, 0), (0, 5)))
            for q in range(4)]


def _pad_rows_q(M):
    return [jnp.pad(M[75 * q:75 * (q + 1)], ((0, 5), (0, 0)))
            for q in range(4)]


def _split_b_q(b):
    return jnp.stack([jnp.pad(b[75 * q:75 * (q + 1)], (0, 5))[None, :]
                      for q in range(4)])


def _mid_w_q(W):
    cols = _pad_cols_q(W)
    return jnp.stack([jnp.stack(_pad_rows_q(c)) for c in cols], axis=1)


def _first_w_q(W):
    cols = _pad_cols_q(W)
    return jnp.stack(
        [jnp.stack([c[64 * qi:64 * (qi + 1)] for qi in range(2)])
         for c in cols], axis=1)


def kernel(x, edge_index, batch,
           W_f0, b_f0, W_f1, b_f1,
           W_c0, b_c0, W_c1, b_c1,
           W_node, b_node, W_edge, b_edge,
           W_b0, b_b0, W_b1, b_b1,
           W_pred, b_pred):
    src = edge_index[0]
    dst = edge_index[1]
    xs = [x[:, :64], x[:, 64:]]

    wf0 = _first_w_q(W_f0)
    wc0 = _first_w_q(W_c0)
    bf0 = _split_b_q(b_f0)
    bc0 = _split_b_q(b_c0)
    wf1 = _mid_w_q(W_f1)
    bf1 = _split_b_q(b_f1)
    wc1 = _mid_w_q(W_c1)
    bc1 = _split_b_q(b_c1)
    wb0 = _mid_w_q(W_b0)
    bb0 = _split_b_q(b_b0)
    wb1 = _mid_w_q(W_b1)
    bb1 = _split_b_q(b_b1)
    wh = jnp.concatenate(
        [W_node, W_edge[:300], W_edge[300:], jnp.zeros((300, 5), jnp.float32)],
        axis=1)
    wh = jnp.stack(_pad_rows_q(wh))
    bh = jnp.array([[b_node[0], 0.0, b_edge[0], 0.0, 0.0, 0.0, 0.0, 0.0]],
                   jnp.float32)
    wpred = jnp.stack(_pad_rows_q(jnp.pad(W_pred, ((0, 0), (0, G - 10)))))
    bpred = jnp.pad(b_pred, (0, G - 10))[None, :]

    ax = _segsum_sc(xs, src, dst, Dq=64, C=400)
    f0c0 = _first_layers(ax, xs, wf0, bf0, wc0, bc0)
    h, c0 = f0c0[:4], f0c0[4:]

    ah = _segsum_sc(h, src, dst)
    xe = _mid_layer(ah, h, wf1, bf1)

    ac = _segsum_sc(c0, src, dst)
    c1_out = _mid_layer(ac, c0, wc1, bc1, wh, bh)
    heads = c1_out[4]
    nc = heads[:, 0:1]
    a_s = heads[:, 1]
    a_d = heads[:, 2]

    ec, psum, pcnt = _edge_head_sc(a_s, a_d, src, dst)

    aC = _segsum_sc(xe, src, dst, w=ec)
    aU = _segsum_sc(xe, src, dst)
    b0 = _masked_layers(aC, aU, xe, xe, nc, wb0, bb0, env_diff=True)
    h1c, h1e = b0[:4], b0[4:]

    aC2 = _segsum_sc(h1c, src, dst, w=ec)
    aE2 = _segsum_sc(h1e, src, dst, w=ec, invert=True)
    b1 = _masked_layers(aC2, aE2, h1c, h1e, nc, wb1, bb1)
    h2c, h2e = b1[:4], b1[4:]

    outs = _pool_predict(h2c, h2e, batch[:, None], nc, wpred, bpred)
    nsum, pC, pE, pA = outs[9], outs[10], outs[11], outs[12]

    nk = nsum[0, 0]
    rn = nsum[0, 1] / N
    ek = jnp.sum(psum)
    re = jnp.sum(pcnt) / E
    ne = N - nk
    ee = E - ek
    cau_loss_reg = (jnp.abs(nk / (nk + ne) - CAU_GAMMA) + (rn - CAU_GAMMA)
                    + jnp.abs(ek / (ek + ee) - CAU_GAMMA) + (re - CAU_GAMMA))

    return pC[:, :10], pE[:, :10], pA[:, :10], cau_loss_reg
